# baseline (speedup 1.0000x reference)
import functools

import jax
import jax.numpy as jnp
from jax.experimental import pallas as pl
from jax.experimental.pallas import tpu as pltpu
from jax.experimental.pallas import tpu_sc as plsc

_E = 16
_K = 8
_DT = 256
_ND = 8
_FT = 128
_NF = 11
_J = 11
_NBG = 6
_NBD = 16
_LA = 4


def _moe_body(x_ref, comb_in_ref, wg_hbm, wu_hbm, wd_hbm,
              wgs_hbm, wus_hbm, wds_hbm, out_ref,
              comb_ref, xbf_ref, g_ref, u_ref, hw_ref,
              wg_buf, wu_buf, wd_buf, sem_g, sem_u, sem_d):
    x = x_ref[...]
    xbf_ref[...] = x.astype(jnp.bfloat16)
    cg = comb_in_ref[...]
    comb_ref[...] = jnp.concatenate([cg[i].T for i in range(cg.shape[0])],
                                    axis=0)
    out_ref[...] = jnp.zeros_like(out_ref)

    total = (_E + 2) * _J

    def _copies(s, fn):
        e = s // _J
        j = s % _J
        jd = jnp.minimum(j, _ND - 1)
        sg = (e * _ND + jd) % _NBG
        sd = s % _NBD

        @pl.when((j < _ND) & (e < _E))
        def _():
            fn(pltpu.make_async_copy(wg_hbm.at[e, jd], wg_buf.at[sg], sem_g.at[sg]))
            fn(pltpu.make_async_copy(wu_hbm.at[e, jd], wu_buf.at[sg], sem_u.at[sg]))

        @pl.when((j < _ND) & (e == _E))
        def _():
            fn(pltpu.make_async_copy(wgs_hbm.at[jd], wg_buf.at[sg], sem_g.at[sg]))
            fn(pltpu.make_async_copy(wus_hbm.at[jd], wu_buf.at[sg], sem_u.at[sg]))

        @pl.when(e < _E)
        def _():
            fn(pltpu.make_async_copy(wd_hbm.at[e, j], wd_buf.at[sd], sem_d.at[sd]))

        @pl.when(e == _E)
        def _():
            fn(pltpu.make_async_copy(wds_hbm.at[j], wd_buf.at[sd], sem_d.at[sd]))

    for s0 in range(_LA):
        _copies(s0, lambda c: c.start())

    def _task(s, carry):
        @pl.when(s + _LA < total)
        def _():
            _copies(s + _LA, lambda c: c.start())

        _copies(s, lambda c: c.wait())

        e = s // _J
        j = s % _J
        jd = jnp.minimum(j, _ND - 1)

        @pl.when((j == 0) & (e >= 1))
        def _activation():
            g = g_ref[...]
            u = u_ref[...]
            h = g * jax.nn.sigmoid(g) * u
            ep = e - 1
            comb = comb_ref[...]
            emask = (jax.lax.broadcasted_iota(jnp.int32, comb.shape, 1)
                     == jnp.minimum(ep, _E - 1))
            we = jnp.sum(jnp.where(emask, comb, 0.0), axis=-1, keepdims=True)
            w = jnp.where(ep < _E, we, 1.0)
            hw = (w * h).astype(jnp.bfloat16)
            for i in range(_NF):
                hw_ref[i] = hw[:, i * _FT:(i + 1) * _FT]

        @pl.when((j < _ND) & (e <= _E))
        def _gate_up():
            sg = (e * _ND + jd) % _NBG
            xs = xbf_ref[:, pl.ds(jd * _DT, _DT)]
            gacc = jnp.dot(xs, wg_buf[sg].astype(jnp.bfloat16),
                           preferred_element_type=jnp.float32)
            uacc = jnp.dot(xs, wu_buf[sg].astype(jnp.bfloat16),
                           preferred_element_type=jnp.float32)
            first = j == 0
            g_ref[...] = jnp.where(first, gacc, g_ref[...] + gacc)
            u_ref[...] = jnp.where(first, uacc, u_ref[...] + uacc)

        @pl.when(e >= 1)
        def _down():
            sd = (s - _J) % _NBD
            out_ref[...] += jnp.dot(hw_ref[j], wd_buf[sd].astype(jnp.bfloat16),
                                    preferred_element_type=jnp.float32)

        return carry

    jax.lax.fori_loop(0, total, _task, 0)


@functools.partial(jax.jit, static_argnames=("interpret",))
def _moe(x, comb, Wg, Wu, Wd, Wg_s, Wu_s, Wd_s, interpret=False):
    T, D = x.shape
    E = comb.shape[1]
    F = Wg.shape[2]

    hbm = pl.BlockSpec(memory_space=pltpu.MemorySpace.HBM)
    vmem = pl.BlockSpec(memory_space=pltpu.MemorySpace.VMEM)

    return pl.pallas_call(
        _moe_body,
        in_specs=[vmem, vmem, hbm, hbm, hbm, hbm, hbm, hbm],
        out_specs=vmem,
        out_shape=jax.ShapeDtypeStruct((T, D), x.dtype),
        scratch_shapes=[
            pltpu.VMEM((T, E), jnp.float32),
            pltpu.VMEM((T, D), jnp.bfloat16),
            pltpu.VMEM((T, F), jnp.float32),
            pltpu.VMEM((T, F), jnp.float32),
            pltpu.VMEM((_NF, T, _FT), jnp.bfloat16),
            pltpu.VMEM((_NBG, _DT, F), jnp.float32),
            pltpu.VMEM((_NBG, _DT, F), jnp.float32),
            pltpu.VMEM((_NBD, _FT, D), jnp.float32),
            pltpu.SemaphoreType.DMA((_NBG,)),
            pltpu.SemaphoreType.DMA((_NBG,)),
            pltpu.SemaphoreType.DMA((_NBD,)),
        ],
        interpret=interpret,
    )(x, comb,
      Wg.reshape(E, _ND, _DT, F), Wu.reshape(E, _ND, _DT, F),
      Wd.reshape(E, _NF, _FT, D),
      Wg_s.reshape(_ND, _DT, F), Wu_s.reshape(_ND, _DT, F),
      Wd_s.reshape(_NF, _FT, D))


def _logits_body(x_ref, wr_ref, br_ref, logits_ref):
    lgt = (jnp.dot(x_ref[...], wr_ref[...],
                   preferred_element_type=jnp.float32) + br_ref[...]).T
    ng = lgt.shape[1] // lgt.shape[0]
    for i in range(ng):
        logits_ref[i] = lgt[:, i * lgt.shape[0]:(i + 1) * lgt.shape[0]]


@jax.jit
def _logits_tc(x, W_router, b_router):
    T = x.shape[0]
    E = W_router.shape[1]
    return pl.pallas_call(
        _logits_body,
        out_shape=jax.ShapeDtypeStruct((T // E, E, E), jnp.float32),
    )(x, W_router, b_router.reshape(1, E))


def _sc_router(logits):
    ntile, E, _ = logits.shape

    mesh = plsc.VectorSubcoreMesh(core_axis_name="c", subcore_axis_name="s")

    @functools.partial(
        pl.kernel, mesh=mesh,
        out_type=jax.ShapeDtypeStruct((ntile, E, E), jnp.float32),
        scratch_types=[pltpu.VMEM((E, E), jnp.float32),
                       pltpu.VMEM((E, E), jnp.float32)],
    )
    def k(logits_hbm, comb_hbm, lbuf, cbuf):
        info = plsc.get_sparse_core_info()
        wid = jax.lax.axis_index("s") * info.num_cores + jax.lax.axis_index("c")

        @pl.when(wid < ntile)
        def _():
            pltpu.sync_copy(logits_hbm.at[wid], lbuf)
            v = [lbuf[e] for e in range(E)]
            p = list(v)
            sel = [None] * E
            neg = jnp.full((E,), -jnp.inf, jnp.float32)
            for r in range(_K):
                mx = p[0]
                for e in range(1, E):
                    mx = jnp.maximum(mx, p[e])
                if r == 0:
                    vmax = mx
                best = jnp.full((E,), E, jnp.int32)
                for e in range(E - 1, -1, -1):
                    best = jnp.where(p[e] == mx, e, best)
                for e in range(E):
                    won = best == e
                    sel[e] = won if sel[e] is None else jnp.logical_or(sel[e], won)
                    p[e] = jnp.where(won, neg, p[e])
            ex = [jnp.where(sel[e], jnp.exp(v[e] - vmax), 0.0) for e in range(E)]
            tot = ex[0]
            for e in range(1, E):
                tot = tot + ex[e]
            for e in range(E):
                cbuf[e] = ex[e] / tot
            pltpu.sync_copy(cbuf, comb_hbm.at[wid])

    return k(logits)


def kernel(x, W_router, b_router, Wg, Wu, Wd, Wg_s, Wu_s, Wd_s):
    logits = _logits_tc(x, W_router, b_router)
    comb = _sc_router(logits)
    return _moe(x, comb, Wg, Wu, Wd, Wg_s, Wu_s, Wd_s)
